# Initial kernel scaffold; baseline (speedup 1.0000x reference)
import jax
import jax.numpy as jnp
from jax.experimental import pallas as pl


def kernel(x, A, W1, b1, W2, b2, W3, b3):
    raise NotImplementedError("write your pallas kernel here")


# speedup vs baseline: 1.0053x; 1.0053x over previous
import jax
import jax.numpy as jnp
from jax.experimental import pallas as pl
from jax.experimental.pallas import tpu as pltpu

_N = 10000
_D = 128
_BM = 400


def _transform_body(x_ref, w_ref, b_ref, o_ref):
    o_ref[...] = (
        jnp.dot(x_ref[...], w_ref[...], preferred_element_type=jnp.float32)
        + b_ref[...]
    )


def _spmm_mid_body(a_ref, g_ref, w_ref, b_ref, o_ref):
    y = jnp.dot(a_ref[...], g_ref[...], preferred_element_type=jnp.float32)
    o_ref[...] = (
        jnp.dot(jnp.maximum(y, 0.0), w_ref[...], preferred_element_type=jnp.float32)
        + b_ref[...]
    )


def _spmm_last_body(a_ref, g_ref, o_ref):
    y = jnp.dot(a_ref[...], g_ref[...], preferred_element_type=jnp.float32)
    denom = jnp.clip(jnp.sum(jnp.abs(y), axis=1, keepdims=True), 1e-12, None)
    o_ref[...] = y / denom


def kernel(x, A, W1, b1, W2, b2, W3, b3):
    f32 = jnp.float32
    g1 = pl.pallas_call(
        _transform_body,
        out_shape=jax.ShapeDtypeStruct((_N, _D), f32),
    )(x, W1, b1[None, :])

    grid = (_N // _BM,)
    a_spec = pl.BlockSpec((_BM, _N), lambda i: (i, 0))
    g_spec = pl.BlockSpec((_N, _D), lambda i: (0, 0))
    w_spec = pl.BlockSpec((_D, _D), lambda i: (0, 0))
    b_spec = pl.BlockSpec((1, _D), lambda i: (0, 0))
    o_spec = pl.BlockSpec((_BM, _D), lambda i: (i, 0))
    params = pltpu.CompilerParams(dimension_semantics=("arbitrary",))

    spmm_mid = pl.pallas_call(
        _spmm_mid_body,
        grid=grid,
        in_specs=[a_spec, g_spec, w_spec, b_spec],
        out_specs=o_spec,
        out_shape=jax.ShapeDtypeStruct((_N, _D), f32),
        compiler_params=params,
    )
    g2 = spmm_mid(A, g1, W2, b2[None, :])
    g3 = spmm_mid(A, g2, W3, b3[None, :])

    h = pl.pallas_call(
        _spmm_last_body,
        grid=grid,
        in_specs=[a_spec, g_spec],
        out_specs=o_spec,
        out_shape=jax.ShapeDtypeStruct((_N, _D), f32),
        compiler_params=params,
    )(A, g3)
    return (h, h, A)
